# baseline (speedup 1.0000x reference)
import functools

import jax
import jax.numpy as jnp
from jax import lax
from jax.experimental import pallas as pl
from jax.experimental.pallas import tpu as pltpu
from jax.experimental.pallas import tpu_sc as plsc

BATCH = 16384
D = 64
NW = 32
PER_W = BATCH // NW
CHUNK = 128
NCHUNK = PER_W // CHUNK
GROUPS = CHUNK // 16


def _mud_body(users, items, uEmbed, itemEmbed, uBias, itemBias, gBias16, price,
              rU, rI, out,
              u_idx, i_idx, uB_v, iB_v, p_v, g_v, out_v,
              uE_b0, iE_b0, rU_b0, rI_b0, uE_b1, iE_b1, rU_b1, rI_b1,
              sem0, sem1, sem_small):
    c = lax.axis_index("c")
    s = lax.axis_index("s")
    wid = s * 2 + c
    base = wid * PER_W

    bufs = ((uE_b0, iE_b0, rU_b0, rI_b0), (uE_b1, iE_b1, rU_b1, rI_b1))
    sems = (sem0, sem1)

    pltpu.sync_copy(users.at[pl.ds(base, PER_W)], u_idx)
    pltpu.sync_copy(items.at[pl.ds(base, PER_W)], i_idx)

    def fire(ch):
        slot = ch % 2
        uidx_c = u_idx.at[pl.ds(ch * CHUNK, CHUNK)]
        iidx_c = i_idx.at[pl.ds(ch * CHUNK, CHUNK)]
        b = bufs[slot]
        sem = sems[slot]
        return (pltpu.async_copy(uEmbed.at[uidx_c], b[0], sem),
                pltpu.async_copy(itemEmbed.at[iidx_c], b[1], sem),
                pltpu.async_copy(rU.at[uidx_c], b[2], sem),
                pltpu.async_copy(rI.at[iidx_c], b[3], sem))

    pend = fire(0)
    d_g = pltpu.async_copy(gBias16, g_v, sem_small)
    d_ub = pltpu.async_copy(uBias.at[u_idx], uB_v, sem_small)
    d_ib = pltpu.async_copy(itemBias.at[i_idx], iB_v, sem_small)
    d_p = pltpu.async_copy(price.at[i_idx], p_v, sem_small)
    d_g.wait()
    d_ub.wait()
    d_ib.wait()
    d_p.wait()

    lane = lax.iota(jnp.int32, 16)

    for ch in range(NCHUNK):
        nxt = fire(ch + 1) if ch + 1 < NCHUNK else None
        for dsc in pend:
            dsc.wait()
        uE_b, iE_b, rU_b, rI_b = bufs[ch % 2]

        def group_body(g, carry, uE_b=uE_b, iE_b=iE_b, rU_b=rU_b,
                       rI_b=rI_b, ch=ch):
            rows = lane + g * 16
            goff = pl.multiple_of(ch * CHUNK + g * 16, 16)
            acc_a = jnp.zeros((16,), jnp.float32)
            acc_r = jnp.zeros((16,), jnp.float32)
            for j in range(D):
                t = (lane + j) & (D - 1)
                ue = plsc.load_gather(uE_b, [rows, t])
                ie = plsc.load_gather(iE_b, [rows, t])
                ru = plsc.load_gather(rU_b, [rows, t])
                ri = plsc.load_gather(rI_b, [rows, t])
                acc_a = acc_a + ue * ie
                acc_r = acc_r + ru * ri
            ub = uB_v[pl.ds(goff, 16)]
            ib = iB_v[pl.ds(goff, 16)]
            pv = p_v[pl.ds(goff, 16)]
            alpha = g_v[...] + ub + ib + acc_a
            e = jnp.exp(-2.0 * jnp.abs(acc_r))
            th = jnp.sign(acc_r) * (1.0 - e) / (1.0 + e)
            res = (0.5 * alpha * th) * (1.0 + jnp.exp(-pv))
            out_v[pl.ds(goff, 16)] = res
            return carry

        lax.fori_loop(0, GROUPS, group_body, 0)
        pend = nxt

    pltpu.sync_copy(out_v, out.at[pl.ds(base, PER_W)])


def kernel(users, items, uEmbed, itemEmbed, uBias, itemBias, gBias, price, rU, rI):
    mesh = plsc.VectorSubcoreMesh(core_axis_name="c", subcore_axis_name="s")
    run = pl.kernel(
        _mud_body,
        out_type=jax.ShapeDtypeStruct((BATCH,), jnp.float32),
        mesh=mesh,
        compiler_params=pltpu.CompilerParams(
            use_tc_tiling_on_sc=False, needs_layout_passes=False
        ),
        scratch_types=[
            pltpu.VMEM((PER_W,), jnp.int32),
            pltpu.VMEM((PER_W,), jnp.int32),
            pltpu.VMEM((PER_W,), jnp.float32),
            pltpu.VMEM((PER_W,), jnp.float32),
            pltpu.VMEM((PER_W,), jnp.float32),
            pltpu.VMEM((16,), jnp.float32),
            pltpu.VMEM((PER_W,), jnp.float32),
            pltpu.VMEM((CHUNK, D), jnp.float32),
            pltpu.VMEM((CHUNK, D), jnp.float32),
            pltpu.VMEM((CHUNK, D), jnp.float32),
            pltpu.VMEM((CHUNK, D), jnp.float32),
            pltpu.VMEM((CHUNK, D), jnp.float32),
            pltpu.VMEM((CHUNK, D), jnp.float32),
            pltpu.VMEM((CHUNK, D), jnp.float32),
            pltpu.VMEM((CHUNK, D), jnp.float32),
            pltpu.SemaphoreType.DMA,
            pltpu.SemaphoreType.DMA,
            pltpu.SemaphoreType.DMA,
        ],
    )
    g16 = jnp.broadcast_to(gBias.reshape(1), (16,))
    return run(users.astype(jnp.int32), items.astype(jnp.int32),
               uEmbed, itemEmbed, uBias.reshape(-1), itemBias.reshape(-1),
               g16, price, rU, rI)


# speedup vs baseline: 1.2807x; 1.2807x over previous
import functools

import jax
import jax.numpy as jnp
from jax import lax
from jax.experimental import pallas as pl
from jax.experimental.pallas import tpu as pltpu
from jax.experimental.pallas import tpu_sc as plsc

BATCH = 16384
D = 64
NW = 32
PER_W = BATCH // NW
CHUNK = 64
NCHUNK = PER_W // CHUNK
GROUPS = CHUNK // 16


def _mud_body(users, items, uEmbed, itemEmbed, uBias, itemBias, gBias16, price,
              rU, rI, out,
              u_idx, i_idx, uB_v, iB_v, p_v, g_v, out_v,
              uE_b0, iE_b0, rU_b0, rI_b0, uE_b1, iE_b1, rU_b1, rI_b1,
              sem0, sem1, sem_small):
    c = lax.axis_index("c")
    s = lax.axis_index("s")
    wid = s * 2 + c
    base = wid * PER_W

    bufs = ((uE_b0, iE_b0, rU_b0, rI_b0), (uE_b1, iE_b1, rU_b1, rI_b1))
    sems = (sem0, sem1)

    pltpu.sync_copy(users.at[pl.ds(base, PER_W)], u_idx)
    pltpu.sync_copy(items.at[pl.ds(base, PER_W)], i_idx)

    lane = lax.iota(jnp.int32, 16)

    def fire(ch, slot):
        b = bufs[slot]
        sem = sems[slot]

        def row_group_body(g, carry):
            goff = ch * CHUNK + g * 16
            u16 = u_idx[pl.ds(goff, 16)]
            i16 = i_idx[pl.ds(goff, 16)]
            for l in range(16):
                k = g * 16 + l
                u = u16[l]
                i = i16[l]
                pltpu.async_copy(uEmbed.at[pl.ds(u, 1), :],
                                 b[0].at[pl.ds(k, 1), :], sem)
                pltpu.async_copy(itemEmbed.at[pl.ds(i, 1), :],
                                 b[1].at[pl.ds(k, 1), :], sem)
                pltpu.async_copy(rU.at[pl.ds(u, 1), :],
                                 b[2].at[pl.ds(k, 1), :], sem)
                pltpu.async_copy(rI.at[pl.ds(i, 1), :],
                                 b[3].at[pl.ds(k, 1), :], sem)
            return carry

        lax.fori_loop(0, CHUNK // 16, row_group_body, 0)

    def drain(slot):
        b = bufs[slot]
        sem = sems[slot]
        pltpu.make_async_copy(uEmbed.at[pl.ds(0, CHUNK), :], b[0], sem).wait()
        pltpu.make_async_copy(itemEmbed.at[pl.ds(0, CHUNK), :], b[1], sem).wait()
        pltpu.make_async_copy(rU.at[pl.ds(0, CHUNK), :], b[2], sem).wait()
        pltpu.make_async_copy(rI.at[pl.ds(0, CHUNK), :], b[3], sem).wait()

    def compute(ch, slot):
        uE_b, iE_b, rU_b, rI_b = bufs[slot]

        def group_body(g, carry):
            rows = lane + g * 16
            goff = ch * CHUNK + g * 16
            acc_a = jnp.zeros((16,), jnp.float32)
            acc_r = jnp.zeros((16,), jnp.float32)
            for j in range(D):
                t = (lane + j) & (D - 1)
                ue = plsc.load_gather(uE_b, [rows, t])
                ie = plsc.load_gather(iE_b, [rows, t])
                ru = plsc.load_gather(rU_b, [rows, t])
                ri = plsc.load_gather(rI_b, [rows, t])
                acc_a = acc_a + ue * ie
                acc_r = acc_r + ru * ri
            ub = uB_v[pl.ds(goff, 16)]
            ib = iB_v[pl.ds(goff, 16)]
            pv = p_v[pl.ds(goff, 16)]
            alpha = g_v[...] + ub + ib + acc_a
            e = jnp.exp(-2.0 * jnp.abs(acc_r))
            th = jnp.sign(acc_r) * (1.0 - e) / (1.0 + e)
            res = (0.5 * alpha * th) * (1.0 + jnp.exp(-pv))
            out_v[pl.ds(goff, 16)] = res
            return carry

        lax.fori_loop(0, GROUPS, group_body, 0)

    fire(0, 0)
    d_g = pltpu.async_copy(gBias16, g_v, sem_small)
    d_ub = pltpu.async_copy(uBias.at[u_idx], uB_v, sem_small)
    d_ib = pltpu.async_copy(itemBias.at[i_idx], iB_v, sem_small)
    d_p = pltpu.async_copy(price.at[i_idx], p_v, sem_small)
    d_g.wait()
    d_ub.wait()
    d_ib.wait()
    d_p.wait()

    def chunk_body(ch, carry):
        even = ch % 2 == 0

        @pl.when(ch + 1 < NCHUNK)
        def _():
            @pl.when(even)
            def _():
                fire(ch + 1, 1)

            @pl.when(jnp.logical_not(even))
            def _():
                fire(ch + 1, 0)

        @pl.when(even)
        def _():
            drain(0)
            compute(ch, 0)

        @pl.when(jnp.logical_not(even))
        def _():
            drain(1)
            compute(ch, 1)

        return carry

    lax.fori_loop(0, NCHUNK, chunk_body, 0)

    pltpu.sync_copy(out_v, out.at[pl.ds(base, PER_W)])


def kernel(users, items, uEmbed, itemEmbed, uBias, itemBias, gBias, price, rU, rI):
    mesh = plsc.VectorSubcoreMesh(core_axis_name="c", subcore_axis_name="s")
    run = pl.kernel(
        _mud_body,
        out_type=jax.ShapeDtypeStruct((BATCH,), jnp.float32),
        mesh=mesh,
        compiler_params=pltpu.CompilerParams(
            use_tc_tiling_on_sc=True, needs_layout_passes=False
        ),
        scratch_types=[
            pltpu.VMEM((PER_W,), jnp.int32),
            pltpu.VMEM((PER_W,), jnp.int32),
            pltpu.VMEM((PER_W,), jnp.float32),
            pltpu.VMEM((PER_W,), jnp.float32),
            pltpu.VMEM((PER_W,), jnp.float32),
            pltpu.VMEM((16,), jnp.float32),
            pltpu.VMEM((PER_W,), jnp.float32),
            pltpu.VMEM((CHUNK, D), jnp.float32),
            pltpu.VMEM((CHUNK, D), jnp.float32),
            pltpu.VMEM((CHUNK, D), jnp.float32),
            pltpu.VMEM((CHUNK, D), jnp.float32),
            pltpu.VMEM((CHUNK, D), jnp.float32),
            pltpu.VMEM((CHUNK, D), jnp.float32),
            pltpu.VMEM((CHUNK, D), jnp.float32),
            pltpu.VMEM((CHUNK, D), jnp.float32),
            pltpu.SemaphoreType.DMA,
            pltpu.SemaphoreType.DMA,
            pltpu.SemaphoreType.DMA,
        ],
    )
    g16 = jnp.broadcast_to(gBias.reshape(1), (16,))
    return run(users.astype(jnp.int32), items.astype(jnp.int32),
               uEmbed, itemEmbed, uBias.reshape(-1), itemBias.reshape(-1),
               g16, price, rU, rI)
